# Initial kernel scaffold; baseline (speedup 1.0000x reference)
import jax
import jax.numpy as jnp
from jax.experimental import pallas as pl


def kernel(predictions, ground_truth, class_weights):
    raise NotImplementedError("write your pallas kernel here")


# speedup vs baseline: 1.8580x; 1.8580x over previous
import functools

import jax
import jax.numpy as jnp
from jax.experimental import pallas as pl
from jax.experimental.pallas import tpu as pltpu

_N = 16384
_C = 32
_CCE_WEIGHT = 0.1
_DICE_WEIGHT = 1.0
_TVERSKY_ALPHA = 0.7
_TVERSKY_BETA = 1.0 - _TVERSKY_ALPHA
_FOCAL_GAMMA = 0.75
_EPS = 1e-08


def _loss_body(x_ref, g_ref, w_ref, out_ref):
    x = x_ref[...]
    g = g_ref[...]
    w = w_ref[...]

    p = jax.nn.softmax(x, axis=1)
    logp = jax.nn.log_softmax(p, axis=1)
    per_sample = -jnp.sum(w * g * logp, axis=1, keepdims=True)
    cce = jnp.sum(per_sample) / _N

    iota = jax.lax.broadcasted_iota(jnp.int32, (_N, _C), 1)
    p_max = jnp.max(p, axis=1, keepdims=True)
    pred_idx = jnp.min(jnp.where(p == p_max, iota, _C), axis=1, keepdims=True)
    g_max = jnp.max(g, axis=1, keepdims=True)
    gt_idx = jnp.min(jnp.where(g == g_max, iota, _C), axis=1, keepdims=True)

    gt_oh = (iota == gt_idx).astype(jnp.float32)
    pred_oh = (iota == pred_idx).astype(jnp.float32)
    row = jnp.sum(gt_oh, axis=0, keepdims=True)
    col = jnp.sum(pred_oh, axis=0, keepdims=True)
    tp = jnp.sum(gt_oh * pred_oh, axis=0, keepdims=True)

    fp = col - tp
    fn = row - tp
    tversky = (tp + _EPS) / (tp + fp * _TVERSKY_BETA + fn * _TVERSKY_ALPHA + _EPS)
    focal = jnp.exp(_FOCAL_GAMMA * jnp.log(jnp.maximum(1.0 - tversky, 1e-30)))
    denom = jnp.sum(row * w)
    wftl = jnp.sum(focal * w) / denom

    out_ref[...] = jnp.reshape(cce * _CCE_WEIGHT + wftl * _DICE_WEIGHT, (1, 1))


def kernel(predictions, ground_truth, class_weights):
    out = pl.pallas_call(
        _loss_body,
        out_shape=jax.ShapeDtypeStruct((1, 1), jnp.float32),
    )(predictions, ground_truth, class_weights.reshape(1, _C))
    return out[0, 0]
